# baseline (speedup 1.0000x reference)
import functools

import numpy as np

import jax
import jax.numpy as jnp
from jax import lax
from jax.experimental import pallas as pl
from jax.experimental.pallas import tpu as pltpu


def _silu(z):
    return z * jax.nn.sigmoid(z)


def _band9(w, W):
    Cin, Cout = w.shape[3], w.shape[4]
    wi = jnp.arange(W)[None, :, None]
    wo = jnp.arange(W)[None, None, :]
    k = jnp.arange(3)[:, None, None]
    E = (wi == wo + k - 1).astype(jnp.bfloat16)
    band = jnp.einsum('kvw,abkic->abviwc', E, w.astype(jnp.bfloat16))
    return band.reshape(9, W * Cin, W * Cout)


def _parity_band(w, W):
    Cin, Cout = w.shape[3], w.shape[4]
    T = jnp.array([[[1., 0., 0.], [0., 1., 1.]],
                   [[1., 1., 0.], [0., 0., 1.]]], w.dtype)
    w2 = jnp.einsum('adi,bej,cfk,ijkmn->abdefmcn', T, T, T, w)
    f = jnp.arange(2)[:, None, None, None]
    wi = jnp.arange(W)[None, :, None, None]
    wo = jnp.arange(W)[None, None, :, None]
    c = jnp.arange(2)[None, None, None, :]
    E2 = (wi == wo + c + f - 1).astype(jnp.bfloat16)
    band = jnp.einsum('fvwc,abdefmcn->abdevmwcn',
                      E2, w2.astype(jnp.bfloat16))
    return band.reshape(16, W * Cin, W * 2 * Cout)


def _group_masks(Cout, tile, num_groups):
    Cg = Cout // num_groups
    m = (jnp.arange(Cout)[:, None] // Cg ==
         jnp.arange(num_groups)[None, :]).astype(jnp.float32)
    m = jnp.tile(m, (tile, 1))
    return m, m.T


def _shift_onehots(S):
    L = np.zeros((9, S * S, S * S), np.float32)
    for t in range(9):
        kd, kh = t // 3, t % 3
        for d in range(S):
            ds = d + kd - 1
            if not 0 <= ds < S:
                continue
            for h in range(S):
                hs = h + kh - 1
                if 0 <= hs < S:
                    L[t, d * S + h, ds * S + hs] = 1.0
    return jnp.asarray(L)


def _scatter_onehots(S, dtype):
    P = np.zeros((4, 4 * S * S, S * S), np.float32)
    for a in range(2):
        for b in range(2):
            for d in range(S):
                for h in range(S):
                    P[a * 2 + b, (2 * d + a) * 2 * S + (2 * h + b),
                      d * S + h] = 1.0
    return jnp.asarray(P, dtype)


def _decoder_body(x_ref, wb1_ref, wb2_ref, wb3_ref, wb4_ref,
                  g2_ref, b2_ref, m2_ref, mt2_ref,
                  g3_ref, b3_ref, m3_ref, mt3_ref,
                  l8_ref, l16_ref, p8_ref, p16_ref,
                  o_ref, *, S):
    D = H = S
    L = 256

    def fdot(a, b):
        return jnp.dot(a, b, preferred_element_type=jnp.float32)

    xb = x_ref[0].astype(jnp.bfloat16)
    acc = jnp.zeros((D * H, L), jnp.float32)
    for t in range(9):
        acc = acc + fdot(l8_ref[t], fdot(xb, wb1_ref[t]))

    x2b = acc.astype(jnp.bfloat16)
    acc = jnp.zeros((D * H, L), jnp.float32)
    for t in range(9):
        acc = acc + fdot(l8_ref[t], fdot(x2b, wb2_ref[t]))
    inv_n2 = 1.0 / (D * H * S * 2)
    s = jnp.sum(acc, axis=0, keepdims=True)
    ss = jnp.sum(acc * acc, axis=0, keepdims=True)
    gmean = fdot(s, m2_ref[...]) * inv_n2
    gmsq = fdot(ss, m2_ref[...]) * inv_n2
    gvar = jnp.maximum(gmsq - gmean * gmean, 0.0)
    ginv = lax.rsqrt(gvar + 1e-5)
    mean_b = fdot(gmean, mt2_ref[...])
    inv_b = fdot(ginv, mt2_ref[...])
    z = (acc - mean_b) * inv_b * g2_ref[...] + b2_ref[...]
    y2 = _silu(z)

    y2b = y2.astype(jnp.bfloat16)
    acc3 = []
    for a in range(2):
        for b in range(2):
            a_ab = jnp.zeros((D * H, L), jnp.float32)
            for kd in range(2):
                for kh in range(2):
                    t = ((a * 2 + b) * 2 + kd) * 2 + kh
                    sel = (a + kd) * 3 + (b + kh)
                    a_ab = a_ab + fdot(l8_ref[sel], fdot(y2b, wb3_ref[t]))
            acc3.append(a_ab)
    inv_n3 = 1.0 / (4 * D * H * S * 2 * 1)
    s = jnp.zeros((1, L), jnp.float32)
    ss = jnp.zeros((1, L), jnp.float32)
    for a_ab in acc3:
        s = s + jnp.sum(a_ab, axis=0, keepdims=True)
        ss = ss + jnp.sum(a_ab * a_ab, axis=0, keepdims=True)
    gmean = fdot(s, m3_ref[...]) * inv_n3
    gmsq = fdot(ss, m3_ref[...]) * inv_n3
    gvar = jnp.maximum(gmsq - gmean * gmean, 0.0)
    ginv = lax.rsqrt(gvar + 1e-5)
    mean_b = fdot(gmean, mt3_ref[...])
    inv_b = fdot(ginv, mt3_ref[...])
    y3 = [_silu((a_ab - mean_b) * inv_b * g3_ref[...] + b3_ref[...])
          for a_ab in acc3]
    x16 = jnp.zeros((4 * D * H, L), jnp.float32)
    for i in range(4):
        x16 = x16 + fdot(p8_ref[i], y3[i])

    x16b = x16.astype(jnp.bfloat16)
    out = jnp.zeros((16 * D * H, L), jnp.float32)
    for a in range(2):
        for b in range(2):
            a_ab = jnp.zeros((4 * D * H, L), jnp.float32)
            for kd in range(2):
                for kh in range(2):
                    t = ((a * 2 + b) * 2 + kd) * 2 + kh
                    sel = (a + kd) * 3 + (b + kh)
                    a_ab = a_ab + fdot(l16_ref[sel], fdot(x16b, wb4_ref[t]))
            out = out + fdot(p16_ref[a * 2 + b],
                             _silu(a_ab).astype(jnp.bfloat16))
    o_ref[0] = out


def kernel(x, w_in, w0, w1, gamma0, gamma1, beta0, beta1, w_out):
    N, D, H, W, Cin = x.shape
    C1 = w_in.shape[-1]
    C2 = w0.shape[-1]
    C3 = w1.shape[-1]
    C4 = w_out.shape[-1]
    S = W
    G = 16

    xf = x.reshape(N, D * H, W * Cin)
    wb1 = _band9(w_in, W)
    wb2 = _band9(w0, W)
    wb3 = _parity_band(w1, W)
    wb4 = _parity_band(w_out, 2 * W)

    g2 = jnp.tile(gamma0.astype(jnp.float32), W).reshape(1, W * C2)
    b2 = jnp.tile(beta0.astype(jnp.float32), W).reshape(1, W * C2)
    m2, mt2 = _group_masks(C2, W, G)
    g3 = jnp.tile(gamma1.astype(jnp.float32), 2 * W).reshape(1, 2 * W * C3)
    b3 = jnp.tile(beta1.astype(jnp.float32), 2 * W).reshape(1, 2 * W * C3)
    m3, mt3 = _group_masks(C3, 2 * W, G)

    l8 = _shift_onehots(S)
    l16 = _shift_onehots(2 * S)
    p8 = _scatter_onehots(S, jnp.float32)
    p16 = _scatter_onehots(2 * S, jnp.bfloat16)

    body = functools.partial(_decoder_body, S=S)
    full = lambda shape: [pl.BlockSpec(shape, (lambda n: (0,) * len(shape)))]
    out = pl.pallas_call(
        body,
        out_shape=jax.ShapeDtypeStruct((N, 16 * D * H, 4 * W * C4),
                                       jnp.float32),
        grid=(N,),
        in_specs=(
            [pl.BlockSpec((1, D * H, W * Cin), lambda n: (n, 0, 0))]
            + full(wb1.shape) + full(wb2.shape) + full(wb3.shape)
            + full(wb4.shape)
            + full(g2.shape) + full(b2.shape) + full(m2.shape)
            + full(mt2.shape)
            + full(g3.shape) + full(b3.shape) + full(m3.shape)
            + full(mt3.shape)
            + full(l8.shape) + full(l16.shape) + full(p8.shape)
            + full(p16.shape)
        ),
        out_specs=pl.BlockSpec((1, 16 * D * H, 4 * W * C4),
                               lambda n: (n, 0, 0)),
        compiler_params=pltpu.CompilerParams(
            dimension_semantics=("parallel",),
            vmem_limit_bytes=64 * 1024 * 1024),
        cost_estimate=pl.CostEstimate(
            flops=2 * N * D * H * (9 * (W * Cin) * (W * C1)
                                   + 9 * (W * C1) * (W * C2)
                                   + 16 * (W * C2) * (2 * W * C3)
                                   + 16 * 4 * (2 * W * C3) * (4 * W * C4)),
            transcendentals=N * 64 * D * H * W * C4 * 2,
            bytes_accessed=4 * N * (D * H * W * Cin
                                    + 64 * D * H * W * C4)),
    )(xf, wb1, wb2, wb3, wb4, g2, b2, m2, mt2, g3, b3, m3, mt3,
      l8, l16, p8, p16)
    return out.reshape(N, 4 * D, 4 * H, 4 * W, C4)


# speedup vs baseline: 1.2958x; 1.2958x over previous
import functools

import jax
import jax.numpy as jnp
from jax import lax
from jax.experimental import pallas as pl
from jax.experimental.pallas import tpu as pltpu


def _silu(z):
    return z * jax.nn.sigmoid(z)


def _band9(w, W):
    Cin, Cout = w.shape[3], w.shape[4]
    wi = jnp.arange(W)[None, :, None]
    wo = jnp.arange(W)[None, None, :]
    k = jnp.arange(3)[:, None, None]
    E = (wi == wo + k - 1).astype(jnp.bfloat16)
    band = jnp.einsum('kvw,abkic->abviwc', E, w.astype(jnp.bfloat16))
    return band.reshape(9, W * Cin, W * Cout)


def _parity_band(w, W):
    Cin, Cout = w.shape[3], w.shape[4]
    T = jnp.array([[[1., 0., 0.], [0., 1., 1.]],
                   [[1., 1., 0.], [0., 0., 1.]]], w.dtype)
    w2 = jnp.einsum('adi,bej,cfk,ijkmn->abdefmcn', T, T, T, w)
    f = jnp.arange(2)[:, None, None, None]
    wi = jnp.arange(W)[None, :, None, None]
    wo = jnp.arange(W)[None, None, :, None]
    c = jnp.arange(2)[None, None, None, :]
    E2 = (wi == wo + c + f - 1).astype(jnp.bfloat16)
    band = jnp.einsum('fvwc,abdefmcn->abdevmwcn',
                      E2, w2.astype(jnp.bfloat16))
    return band.reshape(16, W * Cin, W * 2 * Cout)


def _group_masks(Cout, tile, num_groups):
    Cg = Cout // num_groups
    m = (jnp.arange(Cout)[:, None] // Cg ==
         jnp.arange(num_groups)[None, :]).astype(jnp.float32)
    m = jnp.tile(m, (tile, 1))
    return m, m.T


def _zero_halo(ref, d, h):
    z = jnp.zeros((1, h + 2, ref.shape[-1]), ref.dtype)
    ref[0:1] = z
    ref[d + 1:d + 2] = z
    zc = jnp.zeros((d + 2, 1, ref.shape[-1]), ref.dtype)
    ref[:, 0:1] = zc
    ref[:, h + 1:h + 2] = zc


def _decoder_body(x_ref, wb1_ref, wb2_ref, wb3_ref, wb4_ref,
                  g2_ref, b2_ref, m2_ref, mt2_ref,
                  g3_ref, b3_ref, m3_ref, mt3_ref,
                  o_ref, xpad23_2, xpad4_2, *, S, BN):
    for n in range(BN):
        _one_element(x_ref, wb1_ref, wb2_ref, wb3_ref, wb4_ref,
                     g2_ref, b2_ref, m2_ref, mt2_ref,
                     g3_ref, b3_ref, m3_ref, mt3_ref,
                     o_ref, xpad23_2.at[n], xpad4_2.at[n], n, S)


def _one_element(x_ref, wb1_ref, wb2_ref, wb3_ref, wb4_ref,
                 g2_ref, b2_ref, m2_ref, mt2_ref,
                 g3_ref, b3_ref, m3_ref, mt3_ref,
                 o_ref, xpad23, xpad4, n, S):
    D = H = S
    D2 = H2 = 2 * S
    L = 256

    acc = jnp.zeros((D * H, L), jnp.float32)
    for t in range(9):
        kd, kh = t // 3, t % 3
        lhs = x_ref[n, kd:kd + D, kh:kh + H, :].reshape(D * H,
                                                        x_ref.shape[-1])
        acc = acc + jnp.dot(lhs, wb1_ref[t], preferred_element_type=jnp.float32)

    _zero_halo(xpad23, D, H)
    xpad23[1:D + 1, 1:H + 1, :] = acc.reshape(D, H, L).astype(jnp.bfloat16)
    acc = jnp.zeros((D * H, L), jnp.float32)
    for t in range(9):
        kd, kh = t // 3, t % 3
        lhs = xpad23[kd:kd + D, kh:kh + H, :].reshape(D * H, L)
        acc = acc + jnp.dot(lhs, wb2_ref[t], preferred_element_type=jnp.float32)
    inv_n2 = 1.0 / (D * H * S * 2)
    s = jnp.sum(acc, axis=0, keepdims=True)
    ss = jnp.sum(acc * acc, axis=0, keepdims=True)
    gmean = jnp.dot(s, m2_ref[...], preferred_element_type=jnp.float32) * inv_n2
    gmsq = jnp.dot(ss, m2_ref[...], preferred_element_type=jnp.float32) * inv_n2
    gvar = jnp.maximum(gmsq - gmean * gmean, 0.0)
    ginv = lax.rsqrt(gvar + 1e-5)
    mean_b = jnp.dot(gmean, mt2_ref[...], preferred_element_type=jnp.float32)
    inv_b = jnp.dot(ginv, mt2_ref[...], preferred_element_type=jnp.float32)
    z = (acc - mean_b) * inv_b * g2_ref[...] + b2_ref[...]
    y2 = _silu(z)

    xpad23[1:D + 1, 1:H + 1, :] = y2.reshape(D, H, L).astype(jnp.bfloat16)
    slabs = [[xpad23[i:i + D, j:j + H, :].reshape(D * H, L)
              for j in range(3)] for i in range(3)]
    acc3 = []
    for a in range(2):
        for b in range(2):
            a_ab = jnp.zeros((D * H, L), jnp.float32)
            for kd in range(2):
                for kh in range(2):
                    t = ((a * 2 + b) * 2 + kd) * 2 + kh
                    a_ab = a_ab + jnp.dot(slabs[a + kd][b + kh], wb3_ref[t],
                                          preferred_element_type=jnp.float32)
            acc3.append(a_ab)
    inv_n3 = 1.0 / (4 * D * H * S * 2 * 1)
    s = jnp.zeros((1, L), jnp.float32)
    ss = jnp.zeros((1, L), jnp.float32)
    for a_ab in acc3:
        s = s + jnp.sum(a_ab, axis=0, keepdims=True)
        ss = ss + jnp.sum(a_ab * a_ab, axis=0, keepdims=True)
    gmean = jnp.dot(s, m3_ref[...], preferred_element_type=jnp.float32) * inv_n3
    gmsq = jnp.dot(ss, m3_ref[...], preferred_element_type=jnp.float32) * inv_n3
    gvar = jnp.maximum(gmsq - gmean * gmean, 0.0)
    ginv = lax.rsqrt(gvar + 1e-5)
    mean_b = jnp.dot(gmean, mt3_ref[...], preferred_element_type=jnp.float32)
    inv_b = jnp.dot(ginv, mt3_ref[...], preferred_element_type=jnp.float32)
    y3 = [_silu((a_ab - mean_b) * inv_b * g3_ref[...] + b3_ref[...])
              .reshape(D, H, L).astype(jnp.bfloat16)
          for a_ab in acc3]
    _zero_halo(xpad4, D2, H2)
    u0 = jnp.stack([y3[0], y3[1]], axis=2).reshape(D, H2, L)
    u1 = jnp.stack([y3[2], y3[3]], axis=2).reshape(D, H2, L)
    xpad4[1:D2 + 1, 1:H2 + 1, :] = jnp.stack([u0, u1], axis=1).reshape(
        D2, H2, L)

    slabs4 = [[xpad4[i:i + D2, j:j + H2, :].reshape(D2 * H2, L)
               for j in range(3)] for i in range(3)]
    y4 = []
    for a in range(2):
        for b in range(2):
            a_ab = jnp.zeros((D2 * H2, L), jnp.float32)
            for kd in range(2):
                for kh in range(2):
                    t = ((a * 2 + b) * 2 + kd) * 2 + kh
                    a_ab = a_ab + jnp.dot(slabs4[a + kd][b + kh], wb4_ref[t],
                                          preferred_element_type=jnp.float32)
            y4.append(_silu(a_ab).reshape(D2, H2, L))
    w0_ = jnp.stack([y4[0], y4[1]], axis=2).reshape(D2, 2 * H2, L)
    w1_ = jnp.stack([y4[2], y4[3]], axis=2).reshape(D2, 2 * H2, L)
    o_ref[n] = jnp.stack([w0_, w1_], axis=1).reshape(2 * D2, 2 * H2, L)


def kernel(x, w_in, w0, w1, gamma0, gamma1, beta0, beta1, w_out):
    N, D, H, W, Cin = x.shape
    C1 = w_in.shape[-1]
    C2 = w0.shape[-1]
    C3 = w1.shape[-1]
    C4 = w_out.shape[-1]
    S = W
    G = 16

    xf = jnp.pad(x.reshape(N, D, H, W * Cin).astype(jnp.bfloat16),
                 ((0, 0), (1, 1), (1, 1), (0, 0)))
    wb1 = _band9(w_in, W)
    wb2 = _band9(w0, W)
    wb3 = _parity_band(w1, W)
    wb4 = _parity_band(w_out, 2 * W)

    g2 = jnp.tile(gamma0.astype(jnp.float32), W).reshape(1, W * C2)
    b2 = jnp.tile(beta0.astype(jnp.float32), W).reshape(1, W * C2)
    m2, mt2 = _group_masks(C2, W, G)
    g3 = jnp.tile(gamma1.astype(jnp.float32), 2 * W).reshape(1, 2 * W * C3)
    b3 = jnp.tile(beta1.astype(jnp.float32), 2 * W).reshape(1, 2 * W * C3)
    m3, mt3 = _group_masks(C3, 2 * W, G)

    BN = 2
    body = functools.partial(_decoder_body, S=S, BN=BN)
    out = pl.pallas_call(
        body,
        out_shape=jax.ShapeDtypeStruct((N, 4 * D, 4 * H, 4 * W * C4),
                                       jnp.float32),
        grid=(N // BN,),
        in_specs=[
            pl.BlockSpec((BN, D + 2, H + 2, W * Cin), lambda n: (n, 0, 0, 0)),
            pl.BlockSpec(wb1.shape, lambda n: (0, 0, 0)),
            pl.BlockSpec(wb2.shape, lambda n: (0, 0, 0)),
            pl.BlockSpec(wb3.shape, lambda n: (0, 0, 0)),
            pl.BlockSpec(wb4.shape, lambda n: (0, 0, 0)),
            pl.BlockSpec(g2.shape, lambda n: (0, 0)),
            pl.BlockSpec(b2.shape, lambda n: (0, 0)),
            pl.BlockSpec(m2.shape, lambda n: (0, 0)),
            pl.BlockSpec(mt2.shape, lambda n: (0, 0)),
            pl.BlockSpec(g3.shape, lambda n: (0, 0)),
            pl.BlockSpec(b3.shape, lambda n: (0, 0)),
            pl.BlockSpec(m3.shape, lambda n: (0, 0)),
            pl.BlockSpec(mt3.shape, lambda n: (0, 0)),
        ],
        out_specs=pl.BlockSpec((BN, 4 * D, 4 * H, 4 * W * C4),
                               lambda n: (n, 0, 0, 0)),
        scratch_shapes=[
            pltpu.VMEM((BN, D + 2, H + 2, W * C2), jnp.bfloat16),
            pltpu.VMEM((BN, 2 * D + 2, 2 * H + 2, 2 * W * C3), jnp.bfloat16),
        ],
        compiler_params=pltpu.CompilerParams(
            dimension_semantics=("parallel",),
            vmem_limit_bytes=64 * 1024 * 1024),
        cost_estimate=pl.CostEstimate(
            flops=2 * N * D * H * (9 * (W * Cin) * (W * C1)
                                   + 9 * (W * C1) * (W * C2)
                                   + 16 * (W * C2) * (2 * W * C3)
                                   + 16 * 4 * (2 * W * C3) * (4 * W * C4)),
            transcendentals=N * (4 * D) * (4 * H) * (4 * W) * C4 * 2,
            bytes_accessed=4 * N * (D * H * W * Cin
                                    + 64 * D * H * W * C4)),
    )(xf, wb1, wb2, wb3, wb4, g2, b2, m2, mt2, g3, b3, m3, mt3)
    return out.reshape(N, 4 * D, 4 * H, 4 * W, C4)


# speedup vs baseline: 1.3066x; 1.0083x over previous
import functools

import jax
import jax.numpy as jnp
from jax import lax
from jax.experimental import pallas as pl
from jax.experimental.pallas import tpu as pltpu


def _silu(z):
    return z * jax.nn.sigmoid(z)


def _band9(w, W):
    Cin, Cout = w.shape[3], w.shape[4]
    wi = jnp.arange(W)[None, :, None]
    wo = jnp.arange(W)[None, None, :]
    k = jnp.arange(3)[:, None, None]
    E = (wi == wo + k - 1).astype(jnp.bfloat16)
    band = jnp.einsum('kvw,abkic->abviwc', E, w.astype(jnp.bfloat16))
    return band.reshape(9, W * Cin, W * Cout)


def _parity_band(w, W):
    Cin, Cout = w.shape[3], w.shape[4]
    T = jnp.array([[[1., 0., 0.], [0., 1., 1.]],
                   [[1., 1., 0.], [0., 0., 1.]]], w.dtype)
    w2 = jnp.einsum('adi,bej,cfk,ijkmn->abdefmcn', T, T, T, w)
    f = jnp.arange(2)[:, None, None, None]
    wi = jnp.arange(W)[None, :, None, None]
    wo = jnp.arange(W)[None, None, :, None]
    c = jnp.arange(2)[None, None, None, :]
    E2 = (wi == wo + c + f - 1).astype(jnp.bfloat16)
    band = jnp.einsum('fvwc,abdefmcn->abdevmwcn',
                      E2, w2.astype(jnp.bfloat16))
    return band.reshape(16, W * Cin, W * 2 * Cout)


def _group_masks(Cout, tile, num_groups):
    Cg = Cout // num_groups
    m = (jnp.arange(Cout)[:, None] // Cg ==
         jnp.arange(num_groups)[None, :]).astype(jnp.float32)
    m = jnp.tile(m, (tile, 1))
    return m, m.T


def _scatter_onehots(S):
    import numpy as np
    P = np.zeros((4, 4 * S * S, S * S), np.float32)
    for a in range(2):
        for b in range(2):
            for d in range(S):
                for h in range(S):
                    P[a * 2 + b, (2 * d + a) * 2 * S + (2 * h + b),
                      d * S + h] = 1.0
    return jnp.asarray(P, jnp.bfloat16)


def _zero_halo(ref, d, h):
    z = jnp.zeros((1, h + 2, ref.shape[-1]), ref.dtype)
    ref[0:1] = z
    ref[d + 1:d + 2] = z
    zc = jnp.zeros((d + 2, 1, ref.shape[-1]), ref.dtype)
    ref[:, 0:1] = zc
    ref[:, h + 1:h + 2] = zc


def _decoder_body(x_ref, wb1_ref, wb2_ref, wb3_ref, wb4_ref,
                  g2_ref, b2_ref, m2_ref, mt2_ref,
                  g3_ref, b3_ref, m3_ref, mt3_ref, p16_ref,
                  o_ref, xpad23_2, xpad4_2, *, S, BN):
    for n in range(BN):
        _one_element(x_ref, wb1_ref, wb2_ref, wb3_ref, wb4_ref,
                     g2_ref, b2_ref, m2_ref, mt2_ref,
                     g3_ref, b3_ref, m3_ref, mt3_ref, p16_ref,
                     o_ref, xpad23_2.at[n], xpad4_2.at[n], n, S)


def _one_element(x_ref, wb1_ref, wb2_ref, wb3_ref, wb4_ref,
                 g2_ref, b2_ref, m2_ref, mt2_ref,
                 g3_ref, b3_ref, m3_ref, mt3_ref, p16_ref,
                 o_ref, xpad23, xpad4, n, S):
    D = H = S
    D2 = H2 = 2 * S
    L = 256

    acc = jnp.zeros((D * H, L), jnp.float32)
    for t in range(9):
        kd, kh = t // 3, t % 3
        lhs = x_ref[n, kd:kd + D, kh:kh + H, :].reshape(D * H,
                                                        x_ref.shape[-1])
        acc = acc + jnp.dot(lhs, wb1_ref[t], preferred_element_type=jnp.float32)

    _zero_halo(xpad23, D, H)
    xpad23[1:D + 1, 1:H + 1, :] = acc.reshape(D, H, L).astype(jnp.bfloat16)
    acc = jnp.zeros((D * H, L), jnp.float32)
    for t in range(9):
        kd, kh = t // 3, t % 3
        lhs = xpad23[kd:kd + D, kh:kh + H, :].reshape(D * H, L)
        acc = acc + jnp.dot(lhs, wb2_ref[t], preferred_element_type=jnp.float32)
    inv_n2 = 1.0 / (D * H * S * 2)
    s = jnp.sum(acc, axis=0, keepdims=True)
    ss = jnp.sum(acc * acc, axis=0, keepdims=True)
    gmean = jnp.dot(s, m2_ref[...], preferred_element_type=jnp.float32) * inv_n2
    gmsq = jnp.dot(ss, m2_ref[...], preferred_element_type=jnp.float32) * inv_n2
    gvar = jnp.maximum(gmsq - gmean * gmean, 0.0)
    ginv = lax.rsqrt(gvar + 1e-5)
    mean_b = jnp.dot(gmean, mt2_ref[...], preferred_element_type=jnp.float32)
    inv_b = jnp.dot(ginv, mt2_ref[...], preferred_element_type=jnp.float32)
    z = (acc - mean_b) * inv_b * g2_ref[...] + b2_ref[...]
    y2 = _silu(z)

    xpad23[1:D + 1, 1:H + 1, :] = y2.reshape(D, H, L).astype(jnp.bfloat16)
    slabs = [[xpad23[i:i + D, j:j + H, :].reshape(D * H, L)
              for j in range(3)] for i in range(3)]
    acc3 = []
    for a in range(2):
        for b in range(2):
            a_ab = jnp.zeros((D * H, L), jnp.float32)
            for kd in range(2):
                for kh in range(2):
                    t = ((a * 2 + b) * 2 + kd) * 2 + kh
                    a_ab = a_ab + jnp.dot(slabs[a + kd][b + kh], wb3_ref[t],
                                          preferred_element_type=jnp.float32)
            acc3.append(a_ab)
    inv_n3 = 1.0 / (4 * D * H * S * 2 * 1)
    s = jnp.zeros((1, L), jnp.float32)
    ss = jnp.zeros((1, L), jnp.float32)
    for a_ab in acc3:
        s = s + jnp.sum(a_ab, axis=0, keepdims=True)
        ss = ss + jnp.sum(a_ab * a_ab, axis=0, keepdims=True)
    gmean = jnp.dot(s, m3_ref[...], preferred_element_type=jnp.float32) * inv_n3
    gmsq = jnp.dot(ss, m3_ref[...], preferred_element_type=jnp.float32) * inv_n3
    gvar = jnp.maximum(gmsq - gmean * gmean, 0.0)
    ginv = lax.rsqrt(gvar + 1e-5)
    mean_b = jnp.dot(gmean, mt3_ref[...], preferred_element_type=jnp.float32)
    inv_b = jnp.dot(ginv, mt3_ref[...], preferred_element_type=jnp.float32)
    y3 = [_silu((a_ab - mean_b) * inv_b * g3_ref[...] + b3_ref[...])
              .reshape(D, H, L).astype(jnp.bfloat16)
          for a_ab in acc3]
    _zero_halo(xpad4, D2, H2)
    u0 = jnp.stack([y3[0], y3[1]], axis=2).reshape(D, H2, L)
    u1 = jnp.stack([y3[2], y3[3]], axis=2).reshape(D, H2, L)
    xpad4[1:D2 + 1, 1:H2 + 1, :] = jnp.stack([u0, u1], axis=1).reshape(
        D2, H2, L)

    slabs4 = [[xpad4[i:i + D2, j:j + H2, :].reshape(D2 * H2, L)
               for j in range(3)] for i in range(3)]
    y4 = []
    for a in range(2):
        for b in range(2):
            a_ab = jnp.zeros((D2 * H2, L), jnp.float32)
            for kd in range(2):
                for kh in range(2):
                    t = ((a * 2 + b) * 2 + kd) * 2 + kh
                    a_ab = a_ab + jnp.dot(slabs4[a + kd][b + kh], wb4_ref[t],
                                          preferred_element_type=jnp.float32)
            y4.append(_silu(a_ab).astype(jnp.bfloat16))
    out = jnp.zeros((4 * D2 * H2, L), jnp.float32)
    for i in range(4):
        out = out + jnp.dot(p16_ref[i], y4[i],
                            preferred_element_type=jnp.float32)
    o_ref[n] = out.reshape(2 * D2, 2 * H2, L)


def kernel(x, w_in, w0, w1, gamma0, gamma1, beta0, beta1, w_out):
    N, D, H, W, Cin = x.shape
    C1 = w_in.shape[-1]
    C2 = w0.shape[-1]
    C3 = w1.shape[-1]
    C4 = w_out.shape[-1]
    S = W
    G = 16

    xf = jnp.pad(x.reshape(N, D, H, W * Cin).astype(jnp.bfloat16),
                 ((0, 0), (1, 1), (1, 1), (0, 0)))
    wb1 = _band9(w_in, W)
    wb2 = _band9(w0, W)
    wb3 = _parity_band(w1, W)
    wb4 = _parity_band(w_out, 2 * W)

    g2 = jnp.tile(gamma0.astype(jnp.float32), W).reshape(1, W * C2)
    b2 = jnp.tile(beta0.astype(jnp.float32), W).reshape(1, W * C2)
    m2, mt2 = _group_masks(C2, W, G)
    g3 = jnp.tile(gamma1.astype(jnp.float32), 2 * W).reshape(1, 2 * W * C3)
    b3 = jnp.tile(beta1.astype(jnp.float32), 2 * W).reshape(1, 2 * W * C3)
    m3, mt3 = _group_masks(C3, 2 * W, G)
    p16 = _scatter_onehots(2 * W)

    BN = 2
    body = functools.partial(_decoder_body, S=S, BN=BN)
    out = pl.pallas_call(
        body,
        out_shape=jax.ShapeDtypeStruct((N, 4 * D, 4 * H, 4 * W * C4),
                                       jnp.float32),
        grid=(N // BN,),
        in_specs=[
            pl.BlockSpec((BN, D + 2, H + 2, W * Cin), lambda n: (n, 0, 0, 0)),
            pl.BlockSpec(wb1.shape, lambda n: (0, 0, 0)),
            pl.BlockSpec(wb2.shape, lambda n: (0, 0, 0)),
            pl.BlockSpec(wb3.shape, lambda n: (0, 0, 0)),
            pl.BlockSpec(wb4.shape, lambda n: (0, 0, 0)),
            pl.BlockSpec(g2.shape, lambda n: (0, 0)),
            pl.BlockSpec(b2.shape, lambda n: (0, 0)),
            pl.BlockSpec(m2.shape, lambda n: (0, 0)),
            pl.BlockSpec(mt2.shape, lambda n: (0, 0)),
            pl.BlockSpec(g3.shape, lambda n: (0, 0)),
            pl.BlockSpec(b3.shape, lambda n: (0, 0)),
            pl.BlockSpec(m3.shape, lambda n: (0, 0)),
            pl.BlockSpec(mt3.shape, lambda n: (0, 0)),
            pl.BlockSpec(p16.shape, lambda n: (0, 0, 0)),
        ],
        out_specs=pl.BlockSpec((BN, 4 * D, 4 * H, 4 * W * C4),
                               lambda n: (n, 0, 0, 0)),
        scratch_shapes=[
            pltpu.VMEM((BN, D + 2, H + 2, W * C2), jnp.bfloat16),
            pltpu.VMEM((BN, 2 * D + 2, 2 * H + 2, 2 * W * C3), jnp.bfloat16),
        ],
        compiler_params=pltpu.CompilerParams(
            dimension_semantics=("parallel",),
            vmem_limit_bytes=64 * 1024 * 1024),
        cost_estimate=pl.CostEstimate(
            flops=2 * N * D * H * (9 * (W * Cin) * (W * C1)
                                   + 9 * (W * C1) * (W * C2)
                                   + 16 * (W * C2) * (2 * W * C3)
                                   + 16 * 4 * (2 * W * C3) * (4 * W * C4)),
            transcendentals=N * (4 * D) * (4 * H) * (4 * W) * C4 * 2,
            bytes_accessed=4 * N * (D * H * W * Cin
                                    + 64 * D * H * W * C4)),
    )(xf, wb1, wb2, wb3, wb4, g2, b2, m2, mt2, g3, b3, m3, mt3, p16)
    return out.reshape(N, 4 * D, 4 * H, 4 * W, C4)


# speedup vs baseline: 1.6540x; 1.2659x over previous
import functools

import numpy as np

import jax
import jax.numpy as jnp
from jax import lax
from jax.experimental import pallas as pl
from jax.experimental.pallas import tpu as pltpu


def _silu(z):
    return z * jax.nn.sigmoid(z)


def _band9(w, W):
    Cin, Cout = w.shape[3], w.shape[4]
    wi = jnp.arange(W)[None, :, None]
    wo = jnp.arange(W)[None, None, :]
    k = jnp.arange(3)[:, None, None]
    E = (wi == wo + k - 1).astype(jnp.bfloat16)
    band = jnp.einsum('kvw,abkic->abviwc', E, w.astype(jnp.bfloat16))
    return band.reshape(9, W * Cin, W * Cout)


def _parity_band(w, W):
    Cin, Cout = w.shape[3], w.shape[4]
    T = jnp.array([[[1., 0., 0.], [0., 1., 1.]],
                   [[1., 1., 0.], [0., 0., 1.]]], w.dtype)
    w2 = jnp.einsum('adi,bej,cfk,ijkmn->abdefmcn', T, T, T, w)
    f = jnp.arange(2)[:, None, None, None]
    wi = jnp.arange(W)[None, :, None, None]
    wo = jnp.arange(W)[None, None, :, None]
    c = jnp.arange(2)[None, None, None, :]
    E2 = (wi == wo + c + f - 1).astype(jnp.bfloat16)
    band = jnp.einsum('fvwc,abdefmcn->abdevmwcn',
                      E2, w2.astype(jnp.bfloat16))
    return band.reshape(16, W * Cin, W * 2 * Cout)


def _group_masks(Cout, tile, num_groups):
    Cg = Cout // num_groups
    m = (jnp.arange(Cout)[:, None] // Cg ==
         jnp.arange(num_groups)[None, :]).astype(jnp.float32)
    m = jnp.tile(m, (tile, 1))
    return m, m.T


def _block_mask(BN, R):
    bm = np.zeros((BN, BN * R), np.float32)
    for n in range(BN):
        bm[n, n * R:(n + 1) * R] = 1.0
    return jnp.asarray(bm)


def _zero_halo(ref, d, h):
    bn = ref.shape[0]
    z = jnp.zeros((bn, 1, h + 2, ref.shape[-1]), ref.dtype)
    ref[:, 0:1] = z
    ref[:, d + 1:d + 2] = z
    zc = jnp.zeros((bn, d + 2, 1, ref.shape[-1]), ref.dtype)
    ref[:, :, 0:1] = zc
    ref[:, :, h + 1:h + 2] = zc


def _decoder_body(x_ref, wb1_ref, wb2_ref, wb3_ref, wb4_ref,
                  g2_ref, b2_ref, m2_ref, mt2_ref,
                  g3_ref, b3_ref, m3_ref, mt3_ref, bm_ref,
                  o_ref, xpad23, xpad4, *, S, BN):
    D = H = S
    D2 = H2 = 2 * S
    R = D * H
    R4 = D2 * H2
    M = BN * R
    M4 = BN * R4
    L = 256

    def fdot(a, b):
        return jnp.dot(a, b, preferred_element_type=jnp.float32)

    acc = jnp.zeros((M, L), jnp.float32)
    for t in range(9):
        kd, kh = t // 3, t % 3
        lhs = jnp.concatenate(
            [x_ref[n, kd:kd + D, kh:kh + H, :].reshape(R, x_ref.shape[-1])
             for n in range(BN)], axis=0)
        acc = acc + fdot(lhs, wb1_ref[t])

    _zero_halo(xpad23, D, H)
    xpad23[:, 1:D + 1, 1:H + 1, :] = acc.reshape(BN, D, H, L).astype(
        jnp.bfloat16)
    acc = jnp.zeros((M, L), jnp.float32)
    for t in range(9):
        kd, kh = t // 3, t % 3
        lhs = jnp.concatenate(
            [xpad23[n, kd:kd + D, kh:kh + H, :].reshape(R, L)
             for n in range(BN)], axis=0)
        acc = acc + fdot(lhs, wb2_ref[t])
    inv_n2 = 1.0 / (R * S * 2)
    s = fdot(bm_ref[...], acc)
    ss = fdot(bm_ref[...], acc * acc)
    gmean = fdot(s, m2_ref[...]) * inv_n2
    gmsq = fdot(ss, m2_ref[...]) * inv_n2
    gvar = jnp.maximum(gmsq - gmean * gmean, 0.0)
    ginv = lax.rsqrt(gvar + 1e-5)
    mean_b = fdot(gmean, mt2_ref[...]).reshape(BN, 1, L)
    inv_b = fdot(ginv, mt2_ref[...]).reshape(BN, 1, L)
    z = (acc.reshape(BN, R, L) - mean_b) * inv_b * g2_ref[...] + b2_ref[...]
    y2 = _silu(z)

    xpad23[:, 1:D + 1, 1:H + 1, :] = y2.reshape(BN, D, H, L).astype(
        jnp.bfloat16)
    slabs = [[jnp.concatenate(
                  [xpad23[n, i:i + D, j:j + H, :].reshape(R, L)
                   for n in range(BN)], axis=0)
              for j in range(3)] for i in range(3)]
    acc3 = []
    for a in range(2):
        for b in range(2):
            a_ab = jnp.zeros((M, L), jnp.float32)
            for kd in range(2):
                for kh in range(2):
                    t = ((a * 2 + b) * 2 + kd) * 2 + kh
                    a_ab = a_ab + fdot(slabs[a + kd][b + kh], wb3_ref[t])
            acc3.append(a_ab)
    inv_n3 = 1.0 / (4 * R * S * 2 * 1)
    s = jnp.zeros((BN, L), jnp.float32)
    ss = jnp.zeros((BN, L), jnp.float32)
    for a_ab in acc3:
        s = s + fdot(bm_ref[...], a_ab)
        ss = ss + fdot(bm_ref[...], a_ab * a_ab)
    gmean = fdot(s, m3_ref[...]) * inv_n3
    gmsq = fdot(ss, m3_ref[...]) * inv_n3
    gvar = jnp.maximum(gmsq - gmean * gmean, 0.0)
    ginv = lax.rsqrt(gvar + 1e-5)
    mean_b = fdot(gmean, mt3_ref[...]).reshape(BN, 1, L)
    inv_b = fdot(ginv, mt3_ref[...]).reshape(BN, 1, L)
    y3 = [_silu((a_ab.reshape(BN, R, L) - mean_b) * inv_b
                * g3_ref[...] + b3_ref[...])
          .reshape(BN, D, H, L).astype(jnp.bfloat16)
          for a_ab in acc3]
    _zero_halo(xpad4, D2, H2)
    u0 = jnp.stack([y3[0], y3[1]], axis=3).reshape(BN, D, H2, L)
    u1 = jnp.stack([y3[2], y3[3]], axis=3).reshape(BN, D, H2, L)
    xpad4[:, 1:D2 + 1, 1:H2 + 1, :] = jnp.stack([u0, u1], axis=2).reshape(
        BN, D2, H2, L)

    slabs4 = [[jnp.concatenate(
                   [xpad4[n, i:i + D2, j:j + H2, :].reshape(R4, L)
                    for n in range(BN)], axis=0)
               for j in range(3)] for i in range(3)]
    y4 = []
    for a in range(2):
        for b in range(2):
            a_ab = jnp.zeros((M4, L), jnp.float32)
            for kd in range(2):
                for kh in range(2):
                    t = ((a * 2 + b) * 2 + kd) * 2 + kh
                    a_ab = a_ab + fdot(slabs4[a + kd][b + kh], wb4_ref[t])
            y4.append(_silu(a_ab).reshape(BN, D2, H2, L))
    w0_ = jnp.stack([y4[0], y4[1]], axis=3).reshape(BN, D2, 2 * H2, L)
    w1_ = jnp.stack([y4[2], y4[3]], axis=3).reshape(BN, D2, 2 * H2, L)
    o_ref[...] = jnp.stack([w0_, w1_], axis=2).reshape(BN, 2 * D2, 2 * H2, L)


def kernel(x, w_in, w0, w1, gamma0, gamma1, beta0, beta1, w_out):
    N, D, H, W, Cin = x.shape
    C1 = w_in.shape[-1]
    C2 = w0.shape[-1]
    C3 = w1.shape[-1]
    C4 = w_out.shape[-1]
    S = W
    G = 16

    xf = jnp.pad(x.reshape(N, D, H, W * Cin).astype(jnp.bfloat16),
                 ((0, 0), (1, 1), (1, 1), (0, 0)))
    wb1 = _band9(w_in, W)
    wb2 = _band9(w0, W)
    wb3 = _parity_band(w1, W)
    wb4 = _parity_band(w_out, 2 * W)

    g2 = jnp.tile(gamma0.astype(jnp.float32), W).reshape(1, W * C2)
    b2 = jnp.tile(beta0.astype(jnp.float32), W).reshape(1, W * C2)
    m2, mt2 = _group_masks(C2, W, G)
    g3 = jnp.tile(gamma1.astype(jnp.float32), 2 * W).reshape(1, 2 * W * C3)
    b3 = jnp.tile(beta1.astype(jnp.float32), 2 * W).reshape(1, 2 * W * C3)
    m3, mt3 = _group_masks(C3, 2 * W, G)

    BN = 4
    bm = _block_mask(BN, D * H)

    body = functools.partial(_decoder_body, S=S, BN=BN)
    out = pl.pallas_call(
        body,
        out_shape=jax.ShapeDtypeStruct((N, 4 * D, 4 * H, 4 * W * C4),
                                       jnp.float32),
        grid=(N // BN,),
        in_specs=[
            pl.BlockSpec((BN, D + 2, H + 2, W * Cin), lambda n: (n, 0, 0, 0)),
            pl.BlockSpec(wb1.shape, lambda n: (0, 0, 0)),
            pl.BlockSpec(wb2.shape, lambda n: (0, 0, 0)),
            pl.BlockSpec(wb3.shape, lambda n: (0, 0, 0)),
            pl.BlockSpec(wb4.shape, lambda n: (0, 0, 0)),
            pl.BlockSpec(g2.shape, lambda n: (0, 0)),
            pl.BlockSpec(b2.shape, lambda n: (0, 0)),
            pl.BlockSpec(m2.shape, lambda n: (0, 0)),
            pl.BlockSpec(mt2.shape, lambda n: (0, 0)),
            pl.BlockSpec(g3.shape, lambda n: (0, 0)),
            pl.BlockSpec(b3.shape, lambda n: (0, 0)),
            pl.BlockSpec(m3.shape, lambda n: (0, 0)),
            pl.BlockSpec(mt3.shape, lambda n: (0, 0)),
            pl.BlockSpec(bm.shape, lambda n: (0, 0)),
        ],
        out_specs=pl.BlockSpec((BN, 4 * D, 4 * H, 4 * W * C4),
                               lambda n: (n, 0, 0, 0)),
        scratch_shapes=[
            pltpu.VMEM((BN, D + 2, H + 2, W * C2), jnp.bfloat16),
            pltpu.VMEM((BN, 2 * D + 2, 2 * H + 2, 2 * W * C3), jnp.bfloat16),
        ],
        compiler_params=pltpu.CompilerParams(
            dimension_semantics=("parallel",),
            vmem_limit_bytes=64 * 1024 * 1024),
        cost_estimate=pl.CostEstimate(
            flops=2 * N * D * H * (9 * (W * Cin) * (W * C1)
                                   + 9 * (W * C1) * (W * C2)
                                   + 16 * (W * C2) * (2 * W * C3)
                                   + 16 * 4 * (2 * W * C3) * (4 * W * C4)),
            transcendentals=N * 64 * D * H * W * C4 * 2,
            bytes_accessed=4 * N * (D * H * W * Cin
                                    + 64 * D * H * W * C4)),
    )(xf, wb1, wb2, wb3, wb4, g2, b2, m2, mt2, g3, b3, m3, mt3, bm)
    return out.reshape(N, 4 * D, 4 * H, 4 * W, C4)
